# baseline (speedup 1.0000x reference)
import functools

import jax
import jax.numpy as jnp
from jax import lax
from jax.experimental import pallas as pl
from jax.experimental.pallas import tpu as pltpu
from jax.experimental.pallas import tpu_sc as plsc

N = 10000
D = 128
E = 320000

NC = 2
NS = 16
NW = NC * NS
EDGES_PER_TILE = E // NW
CHUNK = 40
NBUF = 5
EPT_PAD = -EDGES_PER_TILE % (NBUF * CHUNK)
EPT_P = EDGES_PER_TILE + EPT_PAD
N_ITERS = EPT_P // CHUNK
NGROUPS = N_ITERS // NBUF
ROWS_PER_TILE = 624
TAIL_ROWS = N - ROWS_PER_TILE * NS
ONES_PAD = -(-CHUNK // 16) * 16


def _make_sc_agg_body(with_cnt):
    def body(x_hbm, src_hbm, dst_hbm, z2_hbm, *rest):
        if with_cnt:
            (s_out, cnt_out, src_idx0, src_idx1, dst_idx, rows, ones_v,
             cbuf, *sems, s_acc, cnt_acc) = rest
        else:
            (s_out, src_idx0, src_idx1, dst_idx, rows, *sems, s_acc) = rest
        ssrc = sems[0:2]
        sdst = sems[2]
        sg = sems[3:3 + NBUF]
        ss = sems[3 + NBUF:3 + 2 * NBUF]

        c = lax.axis_index("c")
        s = lax.axis_index("s")
        wid = c * NS + s

        if with_cnt:
            for i in range(ONES_PAD // 16):
                ones_v[pl.ds(i * 16, 16)] = jnp.full((16,), 1.0, jnp.float32)
            for i in range(ROWS_PER_TILE // 16):
                cbuf[pl.ds(i * 16, 16)] = jnp.zeros((16,), jnp.float32)

        r0 = pl.multiple_of(s * ROWS_PER_TILE, 8)
        e0 = wid * EPT_P

        def run_pipeline():
            src_idx = (src_idx0, src_idx1)
            pltpu.sync_copy(z2_hbm.at[pl.ds(r0, ROWS_PER_TILE)],
                            s_acc.at[pl.ds(r0, ROWS_PER_TILE)])
            if with_cnt:
                pltpu.sync_copy(cbuf, cnt_acc.at[pl.ds(r0, ROWS_PER_TILE)])

            @pl.when(s == NS - 1)
            def _():
                t0 = ROWS_PER_TILE * NS
                pltpu.sync_copy(z2_hbm.at[pl.ds(t0, TAIL_ROWS)],
                                s_acc.at[pl.ds(t0, TAIL_ROWS)])
                if with_cnt:
                    pltpu.sync_copy(cbuf.at[pl.ds(0, TAIL_ROWS)],
                                    cnt_acc.at[pl.ds(t0, TAIL_ROWS)])

            plsc.subcore_barrier()

            def start_src_load(p, g):
                for b in range(NBUF):
                    base = pl.multiple_of(e0 + (g * NBUF + b) * CHUNK, 8)
                    pltpu.async_copy(src_hbm.at[pl.ds(base, CHUNK)],
                                     src_idx[p].at[b], ssrc[p])

            def wait_src(p):
                for b in range(NBUF):
                    pltpu.make_async_copy(src_hbm.at[pl.ds(0, CHUNK)],
                                          src_idx[p].at[b], ssrc[p]).wait()

            def start_dst_load(g):
                for b in range(NBUF):
                    base = pl.multiple_of(e0 + (g * NBUF + b) * CHUNK, 8)
                    pltpu.async_copy(dst_hbm.at[pl.ds(base, CHUNK)],
                                     dst_idx.at[b], sdst)

            def wait_dst():
                for b in range(NBUF):
                    pltpu.make_async_copy(dst_hbm.at[pl.ds(0, CHUNK)],
                                          dst_idx.at[b], sdst).wait()

            def start_gather(p, b):
                pltpu.async_copy(x_hbm.at[src_idx[p].at[b]], rows.at[b],
                                 sg[b])

            def wait_gather(p, b):
                pltpu.make_async_copy(x_hbm.at[src_idx[p].at[b]],
                                      rows.at[b], sg[b]).wait()

            def do_group(g, p, first=False, last=False):
                for b in range(NBUF):
                    wait_gather(0, b)
                wait_dst()
                descs = []
                for b in range(NBUF):
                    descs.append(pltpu.async_copy(
                        rows.at[b], s_acc.at[dst_idx.at[b]], ss[b],
                        add=True))
                    if with_cnt:
                        descs.append(pltpu.async_copy(
                            ones_v.at[pl.ds(0, CHUNK)],
                            cnt_acc.at[dst_idx.at[b]], ss[b], add=True))
                for d in descs:
                    d.wait()
                if not last:
                    start_src_load(0, g + 1)
                    start_dst_load(g + 1)
                    wait_src(0)
                    for b in range(NBUF):
                        start_gather(0, b)

            start_src_load(0, 0)
            start_dst_load(0)
            wait_src(0)
            for b in range(NBUF):
                start_gather(0, b)

            do_group(0, 0, first=True)

            def pair(g2, carry):
                do_group(2 * g2 + 1, 1)
                do_group(2 * g2 + 2, 0)
                return carry

            lax.fori_loop(0, (NGROUPS - 2) // 2, pair, 0)
            do_group(NGROUPS - 1, 1, last=True)

        run_pipeline()

        plsc.subcore_barrier()

        o0 = pl.multiple_of(c * N + r0, 8)
        pltpu.sync_copy(s_acc.at[pl.ds(r0, ROWS_PER_TILE)],
                        s_out.at[pl.ds(o0, ROWS_PER_TILE)])
        if with_cnt:
            pltpu.sync_copy(cnt_acc.at[pl.ds(r0, ROWS_PER_TILE)], cbuf)
            pltpu.sync_copy(cbuf, cnt_out.at[pl.ds(o0, ROWS_PER_TILE)])

        @pl.when(s == NS - 1)
        def _():
            t0 = ROWS_PER_TILE * NS
            ot = pl.multiple_of(c * N + t0, 8)
            pltpu.sync_copy(s_acc.at[pl.ds(t0, TAIL_ROWS)],
                            s_out.at[pl.ds(ot, TAIL_ROWS)])
            if with_cnt:
                pltpu.sync_copy(cnt_acc.at[pl.ds(t0, TAIL_ROWS)],
                                cbuf.at[pl.ds(0, TAIL_ROWS)])
                pltpu.sync_copy(cbuf.at[pl.ds(0, TAIL_ROWS)],
                                cnt_out.at[pl.ds(ot, TAIL_ROWS)])

    return body


@functools.lru_cache(maxsize=None)
def _sc_agg_kernel(with_cnt):
    out_type = [jax.ShapeDtypeStruct((NC * N, D), jnp.float32)]
    scratch = [
        pltpu.VMEM((NBUF, CHUNK), jnp.int32),
        pltpu.VMEM((NBUF, CHUNK), jnp.int32),
        pltpu.VMEM((NBUF, CHUNK), jnp.int32),
        pltpu.VMEM((NBUF, CHUNK, D), jnp.float32),
    ]
    if with_cnt:
        out_type.append(jax.ShapeDtypeStruct((NC * N,), jnp.float32))
        scratch.append(pltpu.VMEM((ONES_PAD,), jnp.float32))
        scratch.append(pltpu.VMEM((ROWS_PER_TILE,), jnp.float32))
    scratch.extend([pltpu.SemaphoreType.DMA] * (3 + 2 * NBUF))
    scratch.append(pltpu.VMEM_SHARED((N + 8, D), jnp.float32))
    if with_cnt:
        scratch.append(pltpu.VMEM_SHARED((N + 8,), jnp.float32))
    return functools.partial(
        pl.kernel,
        mesh=plsc.VectorSubcoreMesh(core_axis_name="c", subcore_axis_name="s"),
        out_type=out_type,
        scratch_types=scratch,
    )(_make_sc_agg_body(with_cnt))


BLK = 1000


def _combine_body(h_ref, s0_ref, s1_ref, c0_ref, c1_ref, ws_ref, wn_ref,
                  b_ref, o_ref, *, relu):
    cnt = c0_ref[...] + c1_ref[...]
    inv = 1.0 / jnp.maximum(cnt, 1.0)
    agg = (s0_ref[...] + s1_ref[...]) * inv
    acc = jnp.dot(h_ref[...], ws_ref[...], preferred_element_type=jnp.float32)
    acc = acc + jnp.dot(agg, wn_ref[...], preferred_element_type=jnp.float32)
    acc = acc + b_ref[...]
    if relu:
        acc = jnp.maximum(acc, 0.0)
    o_ref[...] = acc


def _combine(h, s2, c2, W_self, W_neigh, b, relu):
    s0 = s2[:N]
    s1 = s2[N:]
    c0 = c2[:N].reshape(N, 1)
    c1 = c2[N:].reshape(N, 1)
    return pl.pallas_call(
        functools.partial(_combine_body, relu=relu),
        grid=(N // BLK,),
        in_specs=[
            pl.BlockSpec((BLK, D), lambda i: (i, 0)),
            pl.BlockSpec((BLK, D), lambda i: (i, 0)),
            pl.BlockSpec((BLK, D), lambda i: (i, 0)),
            pl.BlockSpec((BLK, 1), lambda i: (i, 0)),
            pl.BlockSpec((BLK, 1), lambda i: (i, 0)),
            pl.BlockSpec((D, D), lambda i: (0, 0)),
            pl.BlockSpec((D, D), lambda i: (0, 0)),
            pl.BlockSpec((1, D), lambda i: (0, 0)),
        ],
        out_specs=pl.BlockSpec((BLK, D), lambda i: (i, 0)),
        out_shape=jax.ShapeDtypeStruct((N, D), jnp.float32),
    )(h, s0, s1, c0, c1, W_self, W_neigh, b.reshape(1, D))


def kernel(x, edge_index, W_self1, W_neigh1, b1, W_self2, W_neigh2, b2):
    src3 = jnp.pad(edge_index[0].reshape(NW, EDGES_PER_TILE),
                   ((0, 0), (0, EPT_PAD))).reshape(-1)
    dst3 = jnp.pad(edge_index[1].reshape(NW, EDGES_PER_TILE),
                   ((0, 0), (0, EPT_PAD)), constant_values=N).reshape(-1)
    z2 = jnp.zeros((N, D), jnp.float32)

    agg = _sc_agg_kernel(True)
    s2, c2 = agg(x, src3, dst3, z2)
    h1 = _combine(x, s2, c2, W_self1, W_neigh1, b1, relu=True)

    s2b, _ = agg(h1, src3, dst3, z2)
    out = _combine(h1, s2b, c2, W_self2, W_neigh2, b2, relu=False)
    return out


# speedup vs baseline: 1.1030x; 1.1030x over previous
import functools

import jax
import jax.numpy as jnp
from jax import lax
from jax.experimental import pallas as pl
from jax.experimental.pallas import tpu as pltpu
from jax.experimental.pallas import tpu_sc as plsc

N = 10000
D = 128
E = 320000

NC = 2
NS = 16
NW = NC * NS
EDGES_PER_TILE = E // NW
CHUNK = 40
NBUF = 5
EPT_PAD = -EDGES_PER_TILE % (NBUF * CHUNK)
EPT_P = EDGES_PER_TILE + EPT_PAD
N_ITERS = EPT_P // CHUNK
NGROUPS = N_ITERS // NBUF
ROWS_PER_TILE = 624
TAIL_ROWS = N - ROWS_PER_TILE * NS
ONES_PAD = -(-CHUNK // 16) * 16


def _make_sc_agg_body(with_cnt):
    def body(x_hbm, src_hbm, dst_hbm, z2_hbm, *rest):
        if with_cnt:
            (s_out, cnt_out, src_idx0, src_idx1, dst_idx, rows, ones_v,
             cbuf, *sems, s_acc, cnt_acc) = rest
        else:
            (s_out, src_idx0, src_idx1, dst_idx, rows, *sems, s_acc) = rest
        ssrc = sems[0:2]
        sdst = sems[2]
        sg = sems[3:3 + NBUF]
        ss = sems[3 + NBUF:3 + 2 * NBUF]

        c = lax.axis_index("c")
        s = lax.axis_index("s")
        wid = c * NS + s

        if with_cnt:
            for i in range(ONES_PAD // 16):
                ones_v[pl.ds(i * 16, 16)] = jnp.full((16,), 1.0, jnp.float32)
            for i in range(ROWS_PER_TILE // 16):
                cbuf[pl.ds(i * 16, 16)] = jnp.zeros((16,), jnp.float32)

        r0 = pl.multiple_of(s * ROWS_PER_TILE, 8)
        e0 = wid * EPT_P

        def run_pipeline():
            src_idx = (src_idx0, src_idx1)
            pltpu.sync_copy(z2_hbm.at[pl.ds(r0, ROWS_PER_TILE)],
                            s_acc.at[pl.ds(r0, ROWS_PER_TILE)])
            if with_cnt:
                pltpu.sync_copy(cbuf, cnt_acc.at[pl.ds(r0, ROWS_PER_TILE)])

            @pl.when(s == NS - 1)
            def _():
                t0 = ROWS_PER_TILE * NS
                pltpu.sync_copy(z2_hbm.at[pl.ds(t0, TAIL_ROWS)],
                                s_acc.at[pl.ds(t0, TAIL_ROWS)])
                if with_cnt:
                    pltpu.sync_copy(cbuf.at[pl.ds(0, TAIL_ROWS)],
                                    cnt_acc.at[pl.ds(t0, TAIL_ROWS)])

            plsc.subcore_barrier()

            def start_src_load(p, g):
                for b in range(NBUF):
                    base = pl.multiple_of(e0 + (g * NBUF + b) * CHUNK, 8)
                    pltpu.async_copy(src_hbm.at[pl.ds(base, CHUNK)],
                                     src_idx[p].at[b], ssrc[p])

            def wait_src(p):
                for b in range(NBUF):
                    pltpu.make_async_copy(src_hbm.at[pl.ds(0, CHUNK)],
                                          src_idx[p].at[b], ssrc[p]).wait()

            def start_dst_load(g):
                for b in range(NBUF):
                    base = pl.multiple_of(e0 + (g * NBUF + b) * CHUNK, 8)
                    pltpu.async_copy(dst_hbm.at[pl.ds(base, CHUNK)],
                                     dst_idx.at[b], sdst)

            def wait_dst():
                for b in range(NBUF):
                    pltpu.make_async_copy(dst_hbm.at[pl.ds(0, CHUNK)],
                                          dst_idx.at[b], sdst).wait()

            def start_gather(p, b):
                pltpu.async_copy(x_hbm.at[src_idx[p].at[b]], rows.at[b],
                                 sg[b])

            def wait_gather(p, b):
                pltpu.make_async_copy(x_hbm.at[src_idx[p].at[b]],
                                      rows.at[b], sg[b]).wait()

            def do_group(g, p, first=False, last=False):
                q = 1 - p
                for b in range(NBUF):
                    wait_gather(p, b)
                wait_dst()
                descs = []
                for b in range(NBUF):
                    descs.append(pltpu.async_copy(
                        rows.at[b], s_acc.at[dst_idx.at[b]], ss[b],
                        add=True))
                    if with_cnt:
                        descs.append(pltpu.async_copy(
                            ones_v.at[pl.ds(0, CHUNK)],
                            cnt_acc.at[dst_idx.at[b]], ss[b], add=True))
                for d in descs:
                    d.wait()
                if not last:
                    start_dst_load(g + 1)
                    wait_src(q)
                    for b in range(NBUF):
                        start_gather(q, b)

                    @pl.when(g + 2 < NGROUPS)
                    def _():
                        start_src_load(p, g + 2)

            start_src_load(0, 0)
            start_dst_load(0)
            wait_src(0)
            for b in range(NBUF):
                start_gather(0, b)
            start_src_load(1, 1)

            do_group(0, 0, first=True)

            def pair(g2, carry):
                do_group(2 * g2 + 1, 1)
                do_group(2 * g2 + 2, 0)
                return carry

            lax.fori_loop(0, (NGROUPS - 2) // 2, pair, 0)
            do_group(NGROUPS - 1, 1, last=True)

        run_pipeline()

        plsc.subcore_barrier()

        o0 = pl.multiple_of(c * N + r0, 8)
        pltpu.sync_copy(s_acc.at[pl.ds(r0, ROWS_PER_TILE)],
                        s_out.at[pl.ds(o0, ROWS_PER_TILE)])
        if with_cnt:
            pltpu.sync_copy(cnt_acc.at[pl.ds(r0, ROWS_PER_TILE)], cbuf)
            pltpu.sync_copy(cbuf, cnt_out.at[pl.ds(o0, ROWS_PER_TILE)])

        @pl.when(s == NS - 1)
        def _():
            t0 = ROWS_PER_TILE * NS
            ot = pl.multiple_of(c * N + t0, 8)
            pltpu.sync_copy(s_acc.at[pl.ds(t0, TAIL_ROWS)],
                            s_out.at[pl.ds(ot, TAIL_ROWS)])
            if with_cnt:
                pltpu.sync_copy(cnt_acc.at[pl.ds(t0, TAIL_ROWS)],
                                cbuf.at[pl.ds(0, TAIL_ROWS)])
                pltpu.sync_copy(cbuf.at[pl.ds(0, TAIL_ROWS)],
                                cnt_out.at[pl.ds(ot, TAIL_ROWS)])

    return body


@functools.lru_cache(maxsize=None)
def _sc_agg_kernel(with_cnt):
    out_type = [jax.ShapeDtypeStruct((NC * N, D), jnp.float32)]
    scratch = [
        pltpu.VMEM((NBUF, CHUNK), jnp.int32),
        pltpu.VMEM((NBUF, CHUNK), jnp.int32),
        pltpu.VMEM((NBUF, CHUNK), jnp.int32),
        pltpu.VMEM((NBUF, CHUNK, D), jnp.float32),
    ]
    if with_cnt:
        out_type.append(jax.ShapeDtypeStruct((NC * N,), jnp.float32))
        scratch.append(pltpu.VMEM((ONES_PAD,), jnp.float32))
        scratch.append(pltpu.VMEM((ROWS_PER_TILE,), jnp.float32))
    scratch.extend([pltpu.SemaphoreType.DMA] * (3 + 2 * NBUF))
    scratch.append(pltpu.VMEM_SHARED((N + 8, D), jnp.float32))
    if with_cnt:
        scratch.append(pltpu.VMEM_SHARED((N + 8,), jnp.float32))
    return functools.partial(
        pl.kernel,
        mesh=plsc.VectorSubcoreMesh(core_axis_name="c", subcore_axis_name="s"),
        out_type=out_type,
        scratch_types=scratch,
    )(_make_sc_agg_body(with_cnt))


BLK = 1000


def _combine_body(h_ref, s0_ref, s1_ref, c0_ref, c1_ref, ws_ref, wn_ref,
                  b_ref, o_ref, *, relu):
    cnt = c0_ref[...] + c1_ref[...]
    inv = 1.0 / jnp.maximum(cnt, 1.0)
    agg = (s0_ref[...] + s1_ref[...]) * inv
    acc = jnp.dot(h_ref[...], ws_ref[...], preferred_element_type=jnp.float32)
    acc = acc + jnp.dot(agg, wn_ref[...], preferred_element_type=jnp.float32)
    acc = acc + b_ref[...]
    if relu:
        acc = jnp.maximum(acc, 0.0)
    o_ref[...] = acc


def _combine(h, s2, c2, W_self, W_neigh, b, relu):
    s0 = s2[:N]
    s1 = s2[N:]
    c0 = c2[:N].reshape(N, 1)
    c1 = c2[N:].reshape(N, 1)
    return pl.pallas_call(
        functools.partial(_combine_body, relu=relu),
        grid=(N // BLK,),
        in_specs=[
            pl.BlockSpec((BLK, D), lambda i: (i, 0)),
            pl.BlockSpec((BLK, D), lambda i: (i, 0)),
            pl.BlockSpec((BLK, D), lambda i: (i, 0)),
            pl.BlockSpec((BLK, 1), lambda i: (i, 0)),
            pl.BlockSpec((BLK, 1), lambda i: (i, 0)),
            pl.BlockSpec((D, D), lambda i: (0, 0)),
            pl.BlockSpec((D, D), lambda i: (0, 0)),
            pl.BlockSpec((1, D), lambda i: (0, 0)),
        ],
        out_specs=pl.BlockSpec((BLK, D), lambda i: (i, 0)),
        out_shape=jax.ShapeDtypeStruct((N, D), jnp.float32),
    )(h, s0, s1, c0, c1, W_self, W_neigh, b.reshape(1, D))


def kernel(x, edge_index, W_self1, W_neigh1, b1, W_self2, W_neigh2, b2):
    src3 = jnp.pad(edge_index[0].reshape(NW, EDGES_PER_TILE),
                   ((0, 0), (0, EPT_PAD))).reshape(-1)
    dst3 = jnp.pad(edge_index[1].reshape(NW, EDGES_PER_TILE),
                   ((0, 0), (0, EPT_PAD)), constant_values=N).reshape(-1)
    z2 = jnp.zeros((N, D), jnp.float32)

    agg = _sc_agg_kernel(True)
    s2, c2 = agg(x, src3, dst3, z2)
    h1 = _combine(x, s2, c2, W_self1, W_neigh1, b1, relu=True)

    s2b, _ = agg(h1, src3, dst3, z2)
    out = _combine(h1, s2b, c2, W_self2, W_neigh2, b2, relu=False)
    return out
